# baseline (speedup 1.0000x reference)
import functools

import jax
import jax.numpy as jnp
from jax import lax
from jax.experimental import pallas as pl
from jax.experimental.pallas import tpu as pltpu
from jax.experimental.pallas import tpu_sc as plsc

D_MODEL = 768
N_EXP = 8
D_FF = 3072
TOKENS = 2048
M = 256
T_MAX = 24
TE_W = 64
P = T_MAX * M
CH = 256
NW = 32
TPW = TOKENS // NW


def _route_body(x_ref, gw_ref, gb_ref, logits_ref, pp_ref, ww_ref, te_ref):
    x = x_ref[...]
    logits = jnp.dot(x, gw_ref[...], preferred_element_type=jnp.float32) + gb_ref[...]
    logits_ref[...] = logits
    m = jnp.max(logits, axis=1, keepdims=True)
    ex = jnp.exp(logits - m)
    probs = ex / jnp.sum(ex, axis=1, keepdims=True)
    iota = lax.broadcasted_iota(jnp.int32, (TOKENS, N_EXP), 1)
    p1 = jnp.max(probs, axis=1, keepdims=True)
    i1 = jnp.min(jnp.where(probs == p1, iota, N_EXP), axis=1, keepdims=True)
    sel1 = iota == i1
    probsm = jnp.where(sel1, -1.0, probs)
    p2 = jnp.max(probsm, axis=1, keepdims=True)
    i2 = jnp.min(jnp.where(probsm == p2, iota, N_EXP), axis=1, keepdims=True)
    sel2 = iota == i2
    wpair = jnp.concatenate([p1[:, None, :], p2[:, None, :]], axis=1)
    ww_ref[...] = jnp.broadcast_to(wpair, (TOKENS, 2, 128)).reshape(2 * TOKENS, 128)

    oh0 = sel1.astype(jnp.float32)
    oh1 = sel2.astype(jnp.float32)

    tri = (lax.broadcasted_iota(jnp.int32, (CH, CH), 0)
           > lax.broadcasted_iota(jnp.int32, (CH, CH), 1)).astype(jnp.float32)

    def prefix_excl(oh):
        outs = []
        carry = jnp.zeros((1, N_EXP), jnp.float32)
        for c in range(TOKENS // CH):
            blk = oh[c * CH:(c + 1) * CH]
            outs.append(jnp.dot(tri, blk, preferred_element_type=jnp.float32) + carry)
            carry = carry + jnp.sum(blk, axis=0, keepdims=True)
        return jnp.concatenate(outs, axis=0), carry

    pre0, cnt0 = prefix_excl(oh0)
    pre1, _ = prefix_excl(oh1)
    total = cnt0 + _
    padded = jnp.ceil(total * (1.0 / M)) * M
    tri8x = (lax.broadcasted_iota(jnp.int32, (N_EXP, N_EXP), 0)
             < lax.broadcasted_iota(jnp.int32, (N_EXP, N_EXP), 1)).astype(jnp.float32)
    offs = jnp.dot(padded, tri8x, preferred_element_type=jnp.float32)

    pos0 = jnp.sum(jnp.where(sel1, offs + pre0, 0.0), axis=1, keepdims=True)
    pos1 = jnp.sum(jnp.where(sel2, offs + cnt0 + pre1, 0.0), axis=1, keepdims=True)
    pp_ref[...] = jnp.concatenate([pos0, pos1], axis=1).astype(jnp.int32)

    ntiles = padded * (1.0 / M)
    tri8i = (lax.broadcasted_iota(jnp.int32, (N_EXP, N_EXP), 0)
             <= lax.broadcasted_iota(jnp.int32, (N_EXP, N_EXP), 1)).astype(jnp.float32)
    tend = jnp.dot(ntiles, tri8i, preferred_element_type=jnp.float32)
    iota_t = lax.broadcasted_iota(jnp.int32, (1, TE_W), 1).astype(jnp.float32)
    acc = jnp.zeros((1, TE_W), jnp.float32)
    for e in range(N_EXP):
        acc = acc + (iota_t >= tend[:, e:e + 1]).astype(jnp.float32)
    te = jnp.where(iota_t < tend[:, N_EXP - 1:N_EXP], acc, -1.0)

    present = (total > 0.0).astype(jnp.float32)
    rank = jnp.dot(present, tri8x, preferred_element_type=jnp.float32)
    par_e = rank - 2.0 * jnp.floor(rank * 0.5)
    parity = jnp.zeros((1, TE_W), jnp.float32)
    nxt = jnp.zeros((1, TE_W), jnp.float32)
    for e in range(N_EXP):
        n_e = jnp.float32(-1.0)
        for e2 in range(N_EXP - 1, e, -1):
            n_e = jnp.where(total[0, e2] > 0.0, jnp.float32(e2), n_e)
        sel_e = te == e
        parity = parity + jnp.where(sel_e, par_e[:, e:e + 1], 0.0)
        nxt = nxt + jnp.where(sel_e, n_e, 0.0)
    meta = jnp.concatenate([te, parity, nxt, jnp.zeros((1, TE_W), jnp.float32)], axis=0)
    te_ref[...] = meta.astype(jnp.int32)


def _ffn_body(meta_ref, xs_ref, rw_ref, w1_hbm, b1_ref, w2_hbm, b2_ref, y_ref,
              w1b, w2b, sem1, sem2):
    i = pl.program_id(0)
    te_i = meta_ref[0, i]
    par = meta_ref[1, i]
    nxt = meta_ref[2, i]
    prev = meta_ref[0, jnp.maximum(i - 1, 0)]
    boundary = jnp.logical_or(i == 0, te_i != prev)

    def _start(e, p):
        h1 = D_MODEL // 2
        h2 = D_FF // 2
        pltpu.make_async_copy(w1_hbm.at[e, pl.ds(0, h1)],
                              w1b.at[p, pl.ds(0, h1)], sem1.at[p]).start()
        pltpu.make_async_copy(w1_hbm.at[e, pl.ds(h1, h1)],
                              w1b.at[p, pl.ds(h1, h1)], sem1.at[p]).start()
        pltpu.make_async_copy(w2_hbm.at[e, pl.ds(0, h2)],
                              w2b.at[p, pl.ds(0, h2)], sem2.at[p]).start()
        pltpu.make_async_copy(w2_hbm.at[e, pl.ds(h2, h2)],
                              w2b.at[p, pl.ds(h2, h2)], sem2.at[p]).start()

    @pl.when(jnp.logical_and(te_i >= 0, boundary))
    def _prefetch():
        @pl.when(i == 0)
        def _first():
            _start(te_i, par)

        @pl.when(nxt >= 0)
        def _next():
            _start(nxt, 1 - par)

        pltpu.make_async_copy(w1_hbm.at[0], w1b.at[par], sem1.at[par]).wait()
        pltpu.make_async_copy(w2_hbm.at[0], w2b.at[par], sem2.at[par]).wait()

    @pl.when(te_i >= 0)
    def _compute():
        h = jnp.dot(xs_ref[...], w1b[par], preferred_element_type=jnp.float32) + b1_ref[0]
        h = h * 0.5 * (1.0 + lax.erf(h * 0.7071067811865476))
        o = jnp.dot(h, w2b[par], preferred_element_type=jnp.float32) + b2_ref[0]
        y_ref[...] = o * rw_ref[:, 0:1]


def _dispatch_body(x_hbm, pp_hbm, ww_hbm, xs_hbm, rw_hbm,
                   tok_a, tok_b, ppa, ppb, rwa, rwb, xva, xvb,
                   sga, sgb, ssa, ssb, srw):
    wid = lax.axis_index("s") * 2 + lax.axis_index("c")
    base_tok = wid * TPW
    for j in range(TPW // 16):
        v = lax.iota(jnp.int32, 16) + (16 * j)
        tok_a[pl.ds(16 * j, 16)] = base_tok + (v >> 1)
        tok_b[pl.ds(16 * j, 16)] = base_tok + (TPW // 2) + (v >> 1)
    cp_ga = pltpu.async_copy(x_hbm.at[tok_a], xva, sga)
    cp_gb = pltpu.async_copy(x_hbm.at[tok_b], xvb, sgb)
    pltpu.sync_copy(pp_hbm.at[pl.ds(2 * TPW * wid, TPW)], ppa)
    pltpu.sync_copy(pp_hbm.at[pl.ds(2 * TPW * wid + TPW, TPW)], ppb)
    pltpu.sync_copy(ww_hbm.at[pl.ds(2 * TPW * wid, TPW)], rwa)
    pltpu.sync_copy(ww_hbm.at[pl.ds(2 * TPW * wid + TPW, TPW)], rwb)
    cp_ra = pltpu.async_copy(rwa, rw_hbm.at[ppa], srw)
    cp_rb = pltpu.async_copy(rwb, rw_hbm.at[ppb], srw)
    cp_ga.wait()
    cp_sa = pltpu.async_copy(xva, xs_hbm.at[ppa], ssa)
    cp_gb.wait()
    cp_sb = pltpu.async_copy(xvb, xs_hbm.at[ppb], ssb)
    cp_ra.wait()
    cp_rb.wait()
    cp_sa.wait()
    cp_sb.wait()


_CCH = 4
_CT = TPW // _CCH


def _combine_body(y_hbm, pp_hbm, out_hbm, ppv, g0, g1, ob0, ob1,
                  gs0, gs1, ss0, ss1):
    wid = lax.axis_index("s") * 2 + lax.axis_index("c")
    pltpu.sync_copy(pp_hbm.at[pl.ds(2 * TPW * wid, 2 * TPW)], ppv)
    gbuf = [g0, g1]
    obuf = [ob0, ob1]
    gsem = [gs0, gs1]
    ssem = [ss0, ss1]
    gcp = [None] * _CCH
    scp = [None] * _CCH
    for c in range(2):
        gcp[c] = pltpu.async_copy(
            y_hbm.at[ppv.at[pl.ds(2 * _CT * c, 2 * _CT)]], gbuf[c], gsem[c])
    for c in range(_CCH):
        p = c % 2
        gcp[c].wait()
        if c >= 2:
            scp[c - 2].wait()
        g = gbuf[p]
        ob = obuf[p]

        def body(i, _, g=g, ob=ob):
            for ch in range(D_MODEL // 16):
                a = g[2 * i, pl.ds(16 * ch, 16)]
                b = g[2 * i + 1, pl.ds(16 * ch, 16)]
                ob[i, pl.ds(16 * ch, 16)] = a + b
            return 0

        lax.fori_loop(0, _CT, body, 0)
        scp[c] = pltpu.async_copy(
            ob, out_hbm.at[pl.ds(TPW * wid + _CT * c, _CT)], ssem[p])
        if c + 2 < _CCH:
            gcp[c + 2] = pltpu.async_copy(
                y_hbm.at[ppv.at[pl.ds(2 * _CT * (c + 2), 2 * _CT)]], gbuf[p], gsem[p])
    scp[_CCH - 2].wait()
    scp[_CCH - 1].wait()


@functools.lru_cache(maxsize=1)
def _sc_kernels():
    mesh = plsc.VectorSubcoreMesh(core_axis_name="c", subcore_axis_name="s")
    dispatch = pl.kernel(
        _dispatch_body,
        out_type=(
            jax.ShapeDtypeStruct((P, D_MODEL), jnp.float32),
            jax.ShapeDtypeStruct((P, 128), jnp.float32),
        ),
        mesh=mesh,
        scratch_types=[
            pltpu.VMEM((TPW,), jnp.int32),
            pltpu.VMEM((TPW,), jnp.int32),
            pltpu.VMEM((TPW,), jnp.int32),
            pltpu.VMEM((TPW,), jnp.int32),
            pltpu.VMEM((TPW, 128), jnp.float32),
            pltpu.VMEM((TPW, 128), jnp.float32),
            pltpu.VMEM((TPW, D_MODEL), jnp.float32),
            pltpu.VMEM((TPW, D_MODEL), jnp.float32),
            pltpu.SemaphoreType.DMA,
            pltpu.SemaphoreType.DMA,
            pltpu.SemaphoreType.DMA,
            pltpu.SemaphoreType.DMA,
            pltpu.SemaphoreType.DMA,
        ],
    )
    combine = pl.kernel(
        _combine_body,
        out_type=jax.ShapeDtypeStruct((TOKENS, D_MODEL), jnp.float32),
        mesh=mesh,
        scratch_types=[
            pltpu.VMEM((2 * TPW,), jnp.int32),
            pltpu.VMEM((2 * _CT, D_MODEL), jnp.float32),
            pltpu.VMEM((2 * _CT, D_MODEL), jnp.float32),
            pltpu.VMEM((_CT, D_MODEL), jnp.float32),
            pltpu.VMEM((_CT, D_MODEL), jnp.float32),
            pltpu.SemaphoreType.DMA,
            pltpu.SemaphoreType.DMA,
            pltpu.SemaphoreType.DMA,
            pltpu.SemaphoreType.DMA,
        ],
    )
    return dispatch, combine


@functools.partial(jax.jit, static_argnames=("interpret",))
def kernel(x, gate_w, gate_b, w1, b1, w2, b2, interpret=False):
    batch, seq, d_model = x.shape
    x_flat = x.reshape(batch * seq, d_model)

    logits, pp, ww, te = pl.pallas_call(
        _route_body,
        out_shape=[
            jax.ShapeDtypeStruct((TOKENS, N_EXP), jnp.float32),
            jax.ShapeDtypeStruct((TOKENS, 2), jnp.int32),
            jax.ShapeDtypeStruct((2 * TOKENS, 128), jnp.float32),
            jax.ShapeDtypeStruct((4, TE_W), jnp.int32),
        ],
        interpret=interpret,
    )(x_flat, gate_w, gate_b.reshape(1, N_EXP))

    dispatch, combine = _sc_kernels()
    pp_flat = pp.reshape(-1)
    xs, rw = dispatch(x_flat, pp_flat, ww)

    grid_spec = pltpu.PrefetchScalarGridSpec(
        num_scalar_prefetch=1,
        grid=(T_MAX,),
        in_specs=[
            pl.BlockSpec((M, D_MODEL), lambda i, m: (i, 0)),
            pl.BlockSpec((M, 128), lambda i, m: (i, 0)),
            pl.BlockSpec(memory_space=pltpu.MemorySpace.HBM),
            pl.BlockSpec((1, 1, D_FF), lambda i, m: (jnp.maximum(m[0, i], 0), 0, 0)),
            pl.BlockSpec(memory_space=pltpu.MemorySpace.HBM),
            pl.BlockSpec((1, 1, D_MODEL), lambda i, m: (jnp.maximum(m[0, i], 0), 0, 0)),
        ],
        out_specs=pl.BlockSpec((M, D_MODEL), lambda i, m: (i, 0)),
        scratch_shapes=[
            pltpu.VMEM((2, D_MODEL, D_FF), jnp.float32),
            pltpu.VMEM((2, D_FF, D_MODEL), jnp.float32),
            pltpu.SemaphoreType.DMA((2,)),
            pltpu.SemaphoreType.DMA((2,)),
        ],
    )
    y = pl.pallas_call(
        _ffn_body,
        grid_spec=grid_spec,
        out_shape=jax.ShapeDtypeStruct((P, D_MODEL), jnp.float32),
        compiler_params=pltpu.CompilerParams(dimension_semantics=("arbitrary",)),
        interpret=interpret,
    )(te, xs, rw,
      w1, b1.reshape(N_EXP, 1, D_FF),
      w2, b2.reshape(N_EXP, 1, D_MODEL))

    out = combine(y, pp_flat)
    return out.reshape(batch, seq, d_model), logits


# speedup vs baseline: 1.0167x; 1.0167x over previous
import functools

import jax
import jax.numpy as jnp
from jax import lax
from jax.experimental import pallas as pl
from jax.experimental.pallas import tpu as pltpu
from jax.experimental.pallas import tpu_sc as plsc

D_MODEL = 768
N_EXP = 8
D_FF = 3072
TOKENS = 2048
M = 256
T_MAX = 24
TE_W = 64
P = (T_MAX + 1) * M
CH = 256
NW = 32
TPW = TOKENS // NW


def _route_body(x_ref, gw_ref, gb_ref, logits_ref, pp_ref, ww_ref, te_ref):
    x = x_ref[...]
    logits = jnp.dot(x, gw_ref[...], preferred_element_type=jnp.float32) + gb_ref[...]
    logits_ref[...] = logits
    m = jnp.max(logits, axis=1, keepdims=True)
    ex = jnp.exp(logits - m)
    probs = ex / jnp.sum(ex, axis=1, keepdims=True)
    iota = lax.broadcasted_iota(jnp.int32, (TOKENS, N_EXP), 1)
    p1 = jnp.max(probs, axis=1, keepdims=True)
    i1 = jnp.min(jnp.where(probs == p1, iota, N_EXP), axis=1, keepdims=True)
    sel1 = iota == i1
    probsm = jnp.where(sel1, -1.0, probs)
    p2 = jnp.max(probsm, axis=1, keepdims=True)
    i2 = jnp.min(jnp.where(probsm == p2, iota, N_EXP), axis=1, keepdims=True)
    sel2 = iota == i2
    wpair = jnp.concatenate([p1[:, None, :], p2[:, None, :]], axis=1)
    ww_ref[...] = jnp.broadcast_to(wpair, (TOKENS, 2, 128)).reshape(2 * TOKENS, 128)

    oh0 = sel1.astype(jnp.float32)
    oh1 = sel2.astype(jnp.float32)

    tri = (lax.broadcasted_iota(jnp.int32, (CH, CH), 0)
           > lax.broadcasted_iota(jnp.int32, (CH, CH), 1)).astype(jnp.float32)

    def prefix_excl(oh):
        outs = []
        carry = jnp.zeros((1, N_EXP), jnp.float32)
        for c in range(TOKENS // CH):
            blk = oh[c * CH:(c + 1) * CH]
            outs.append(jnp.dot(tri, blk, preferred_element_type=jnp.float32) + carry)
            carry = carry + jnp.sum(blk, axis=0, keepdims=True)
        return jnp.concatenate(outs, axis=0), carry

    pre0, cnt0 = prefix_excl(oh0)
    pre1, _ = prefix_excl(oh1)
    total = cnt0 + _
    padded = jnp.ceil(total * (1.0 / M)) * M
    tri8x = (lax.broadcasted_iota(jnp.int32, (N_EXP, N_EXP), 0)
             < lax.broadcasted_iota(jnp.int32, (N_EXP, N_EXP), 1)).astype(jnp.float32)
    offs = jnp.dot(padded, tri8x, preferred_element_type=jnp.float32)

    pos0 = jnp.sum(jnp.where(sel1, offs + pre0, 0.0), axis=1, keepdims=True)
    pos1 = jnp.sum(jnp.where(sel2, offs + cnt0 + pre1, 0.0), axis=1, keepdims=True)
    pp_ref[...] = jnp.concatenate([pos0, pos1], axis=1).astype(jnp.int32)

    ntiles = padded * (1.0 / M)
    tri8i = (lax.broadcasted_iota(jnp.int32, (N_EXP, N_EXP), 0)
             <= lax.broadcasted_iota(jnp.int32, (N_EXP, N_EXP), 1)).astype(jnp.float32)
    tend = jnp.dot(ntiles, tri8i, preferred_element_type=jnp.float32)
    iota_t = lax.broadcasted_iota(jnp.int32, (1, TE_W), 1).astype(jnp.float32)
    acc = jnp.zeros((1, TE_W), jnp.float32)
    for e in range(N_EXP):
        acc = acc + (iota_t >= tend[:, e:e + 1]).astype(jnp.float32)
    te = jnp.where(iota_t < tend[:, N_EXP - 1:N_EXP], acc, -1.0)

    present = (total > 0.0).astype(jnp.float32)
    rank = jnp.dot(present, tri8x, preferred_element_type=jnp.float32)
    par_e = rank - 2.0 * jnp.floor(rank * 0.5)
    parity = jnp.zeros((1, TE_W), jnp.float32)
    nxt = jnp.zeros((1, TE_W), jnp.float32)
    for e in range(N_EXP):
        n_e = jnp.float32(-1.0)
        for e2 in range(N_EXP - 1, e, -1):
            n_e = jnp.where(total[0, e2] > 0.0, jnp.float32(e2), n_e)
        sel_e = te == e
        parity = parity + jnp.where(sel_e, par_e[:, e:e + 1], 0.0)
        nxt = nxt + jnp.where(sel_e, n_e, 0.0)
    meta = jnp.concatenate([te, parity, nxt, jnp.zeros((1, TE_W), jnp.float32)], axis=0)
    te_ref[...] = meta.astype(jnp.int32)


def _ffn_body(meta_ref, xs_ref, rw_ref, w1_hbm, b1_ref, w2_hbm, b2_ref, y_ref,
              w1b, w2b, sem1, sem2):
    i = pl.program_id(0)
    te_i = meta_ref[0, i]
    par = meta_ref[1, i]
    nxt = meta_ref[2, i]
    prev = meta_ref[0, jnp.maximum(i - 1, 0)]
    boundary = jnp.logical_or(i == 0, te_i != prev)

    def _start(e, p):
        h1 = D_MODEL // 2
        h2 = D_FF // 2
        pltpu.make_async_copy(w1_hbm.at[e, pl.ds(0, h1)],
                              w1b.at[p, pl.ds(0, h1)], sem1.at[p]).start()
        pltpu.make_async_copy(w1_hbm.at[e, pl.ds(h1, h1)],
                              w1b.at[p, pl.ds(h1, h1)], sem1.at[p]).start()
        pltpu.make_async_copy(w2_hbm.at[e, pl.ds(0, h2)],
                              w2b.at[p, pl.ds(0, h2)], sem2.at[p]).start()
        pltpu.make_async_copy(w2_hbm.at[e, pl.ds(h2, h2)],
                              w2b.at[p, pl.ds(h2, h2)], sem2.at[p]).start()

    @pl.when(jnp.logical_and(te_i >= 0, boundary))
    def _prefetch():
        @pl.when(i == 0)
        def _first():
            _start(te_i, par)

        @pl.when(nxt >= 0)
        def _next():
            _start(nxt, 1 - par)

        pltpu.make_async_copy(w1_hbm.at[0], w1b.at[par], sem1.at[par]).wait()
        pltpu.make_async_copy(w2_hbm.at[0], w2b.at[par], sem2.at[par]).wait()

    @pl.when(te_i >= 0)
    def _compute():
        h = jnp.dot(xs_ref[...], w1b[par], preferred_element_type=jnp.float32) + b1_ref[0]
        h = h * 0.5 * (1.0 + lax.erf(h * 0.7071067811865476))
        o = jnp.dot(h, w2b[par], preferred_element_type=jnp.float32) + b2_ref[0]
        y_ref[...] = o * rw_ref[:, 0:1]


def _dispatch_body(x_hbm, pp_hbm, ww_hbm, xs_hbm, rw_hbm,
                   tok_a, tok_b, ppa, ppb, rwa, rwb, xva, xvb,
                   sga, sgb, ssa, ssb, srw):
    wid = lax.axis_index("s") * 2 + lax.axis_index("c")
    base_tok = wid * TPW
    for j in range(TPW // 16):
        v = lax.iota(jnp.int32, 16) + (16 * j)
        tok_a[pl.ds(16 * j, 16)] = base_tok + (v >> 1)
        tok_b[pl.ds(16 * j, 16)] = base_tok + (TPW // 2) + (v >> 1)
    cp_ga = pltpu.async_copy(x_hbm.at[tok_a], xva, sga)
    cp_gb = pltpu.async_copy(x_hbm.at[tok_b], xvb, sgb)
    pltpu.sync_copy(pp_hbm.at[pl.ds(2 * TPW * wid, TPW)], ppa)
    pltpu.sync_copy(pp_hbm.at[pl.ds(2 * TPW * wid + TPW, TPW)], ppb)
    pltpu.sync_copy(ww_hbm.at[pl.ds(2 * TPW * wid, TPW)], rwa)
    pltpu.sync_copy(ww_hbm.at[pl.ds(2 * TPW * wid + TPW, TPW)], rwb)
    cp_ra = pltpu.async_copy(rwa, rw_hbm.at[ppa], srw)
    cp_rb = pltpu.async_copy(rwb, rw_hbm.at[ppb], srw)
    cp_ga.wait()
    cp_sa = pltpu.async_copy(xva, xs_hbm.at[ppa], ssa)
    cp_gb.wait()
    cp_sb = pltpu.async_copy(xvb, xs_hbm.at[ppb], ssb)
    cp_ra.wait()
    cp_rb.wait()
    cp_sa.wait()
    cp_sb.wait()


_CCH = 4
_CT = TPW // _CCH


def _combine_body(y_hbm, pp_hbm, out_hbm, ppv, g0, g1, ob0, ob1,
                  gs0, gs1, ss0, ss1):
    wid = lax.axis_index("s") * 2 + lax.axis_index("c")
    pltpu.sync_copy(pp_hbm.at[pl.ds(2 * TPW * wid, 2 * TPW)], ppv)
    gbuf = [g0, g1]
    obuf = [ob0, ob1]
    gsem = [gs0, gs1]
    ssem = [ss0, ss1]
    gcp = [None] * _CCH
    scp = [None] * _CCH
    for c in range(2):
        gcp[c] = pltpu.async_copy(
            y_hbm.at[ppv.at[pl.ds(2 * _CT * c, 2 * _CT)]], gbuf[c], gsem[c])
    for c in range(_CCH):
        p = c % 2
        gcp[c].wait()
        if c >= 2:
            scp[c - 2].wait()
        g = gbuf[p]
        ob = obuf[p]

        def body(i, _, g=g, ob=ob):
            for ch in range(D_MODEL // 16):
                a = g[2 * i, pl.ds(16 * ch, 16)]
                b = g[2 * i + 1, pl.ds(16 * ch, 16)]
                ob[i, pl.ds(16 * ch, 16)] = a + b
            return 0

        lax.fori_loop(0, _CT, body, 0)
        scp[c] = pltpu.async_copy(
            ob, out_hbm.at[pl.ds(TPW * wid + _CT * c, _CT)], ssem[p])
        if c + 2 < _CCH:
            gcp[c + 2] = pltpu.async_copy(
                y_hbm.at[ppv.at[pl.ds(2 * _CT * (c + 2), 2 * _CT)]], gbuf[p], gsem[p])
    scp[_CCH - 2].wait()
    scp[_CCH - 1].wait()


@functools.lru_cache(maxsize=1)
def _sc_kernels():
    mesh = plsc.VectorSubcoreMesh(core_axis_name="c", subcore_axis_name="s")
    dispatch = pl.kernel(
        _dispatch_body,
        out_type=(
            jax.ShapeDtypeStruct((P, D_MODEL), jnp.float32),
            jax.ShapeDtypeStruct((P, 128), jnp.float32),
        ),
        mesh=mesh,
        scratch_types=[
            pltpu.VMEM((TPW,), jnp.int32),
            pltpu.VMEM((TPW,), jnp.int32),
            pltpu.VMEM((TPW,), jnp.int32),
            pltpu.VMEM((TPW,), jnp.int32),
            pltpu.VMEM((TPW, 128), jnp.float32),
            pltpu.VMEM((TPW, 128), jnp.float32),
            pltpu.VMEM((TPW, D_MODEL), jnp.float32),
            pltpu.VMEM((TPW, D_MODEL), jnp.float32),
            pltpu.SemaphoreType.DMA,
            pltpu.SemaphoreType.DMA,
            pltpu.SemaphoreType.DMA,
            pltpu.SemaphoreType.DMA,
            pltpu.SemaphoreType.DMA,
        ],
    )
    combine = pl.kernel(
        _combine_body,
        out_type=jax.ShapeDtypeStruct((TOKENS, D_MODEL), jnp.float32),
        mesh=mesh,
        scratch_types=[
            pltpu.VMEM((2 * TPW,), jnp.int32),
            pltpu.VMEM((2 * _CT, D_MODEL), jnp.float32),
            pltpu.VMEM((2 * _CT, D_MODEL), jnp.float32),
            pltpu.VMEM((_CT, D_MODEL), jnp.float32),
            pltpu.VMEM((_CT, D_MODEL), jnp.float32),
            pltpu.SemaphoreType.DMA,
            pltpu.SemaphoreType.DMA,
            pltpu.SemaphoreType.DMA,
            pltpu.SemaphoreType.DMA,
        ],
    )
    return dispatch, combine


@functools.partial(jax.jit, static_argnames=("interpret",))
def kernel(x, gate_w, gate_b, w1, b1, w2, b2, interpret=False):
    batch, seq, d_model = x.shape
    x_flat = x.reshape(batch * seq, d_model)

    logits, pp, ww, te = pl.pallas_call(
        _route_body,
        out_shape=[
            jax.ShapeDtypeStruct((TOKENS, N_EXP), jnp.float32),
            jax.ShapeDtypeStruct((TOKENS, 2), jnp.int32),
            jax.ShapeDtypeStruct((2 * TOKENS, 128), jnp.float32),
            jax.ShapeDtypeStruct((4, TE_W), jnp.int32),
        ],
        interpret=interpret,
    )(x_flat, gate_w, gate_b.reshape(1, N_EXP))

    dispatch, combine = _sc_kernels()
    pp_flat = pp.reshape(-1)
    xs, rw = dispatch(x_flat, pp_flat, ww)

    grid_spec = pltpu.PrefetchScalarGridSpec(
        num_scalar_prefetch=1,
        grid=(T_MAX,),
        in_specs=[
            pl.BlockSpec((M, D_MODEL), lambda i, m: (jnp.where(m[0, i] >= 0, i, T_MAX), 0)),
            pl.BlockSpec((M, 128), lambda i, m: (jnp.where(m[0, i] >= 0, i, T_MAX), 0)),
            pl.BlockSpec(memory_space=pltpu.MemorySpace.HBM),
            pl.BlockSpec((1, 1, D_FF), lambda i, m: (jnp.maximum(m[0, i], 0), 0, 0)),
            pl.BlockSpec(memory_space=pltpu.MemorySpace.HBM),
            pl.BlockSpec((1, 1, D_MODEL), lambda i, m: (jnp.maximum(m[0, i], 0), 0, 0)),
        ],
        out_specs=pl.BlockSpec((M, D_MODEL), lambda i, m: (jnp.where(m[0, i] >= 0, i, T_MAX), 0)),
        scratch_shapes=[
            pltpu.VMEM((2, D_MODEL, D_FF), jnp.float32),
            pltpu.VMEM((2, D_FF, D_MODEL), jnp.float32),
            pltpu.SemaphoreType.DMA((2,)),
            pltpu.SemaphoreType.DMA((2,)),
        ],
    )
    y = pl.pallas_call(
        _ffn_body,
        grid_spec=grid_spec,
        out_shape=jax.ShapeDtypeStruct((P, D_MODEL), jnp.float32),
        compiler_params=pltpu.CompilerParams(dimension_semantics=("arbitrary",)),
        interpret=interpret,
    )(te, xs, rw,
      w1, b1.reshape(N_EXP, 1, D_FF),
      w2, b2.reshape(N_EXP, 1, D_MODEL))

    out = combine(y, pp_flat)
    return out.reshape(batch, seq, d_model), logits


# speedup vs baseline: 1.0177x; 1.0010x over previous
import functools

import jax
import jax.numpy as jnp
from jax import lax
from jax.experimental import pallas as pl
from jax.experimental.pallas import tpu as pltpu
from jax.experimental.pallas import tpu_sc as plsc

D_MODEL = 768
N_EXP = 8
D_FF = 3072
TOKENS = 2048
M = 256
T_MAX = 24
TE_W = 64
P = (T_MAX + 1) * M
CH = 256
NW = 32
TPW = TOKENS // NW


def _route_body(x_ref, gw_ref, gb_ref, logits_ref, pp_ref, ww_ref, te_ref):
    x = x_ref[...]
    logits = jnp.dot(x, gw_ref[...], preferred_element_type=jnp.float32) + gb_ref[...]
    logits_ref[...] = logits
    m = jnp.max(logits, axis=1, keepdims=True)
    ex = jnp.exp(logits - m)
    probs = ex / jnp.sum(ex, axis=1, keepdims=True)
    iota = lax.broadcasted_iota(jnp.int32, (TOKENS, N_EXP), 1)
    p1 = jnp.max(probs, axis=1, keepdims=True)
    i1 = jnp.min(jnp.where(probs == p1, iota, N_EXP), axis=1, keepdims=True)
    sel1 = iota == i1
    probsm = jnp.where(sel1, -1.0, probs)
    p2 = jnp.max(probsm, axis=1, keepdims=True)
    i2 = jnp.min(jnp.where(probsm == p2, iota, N_EXP), axis=1, keepdims=True)
    sel2 = iota == i2
    wpair = jnp.concatenate([p1[:, None, :], p2[:, None, :]], axis=1)
    ww_ref[...] = jnp.broadcast_to(wpair, (TOKENS, 2, 128)).reshape(2 * TOKENS, 128)

    oh0 = sel1.astype(jnp.float32)
    oh1 = sel2.astype(jnp.float32)

    tri = (lax.broadcasted_iota(jnp.int32, (CH, CH), 0)
           > lax.broadcasted_iota(jnp.int32, (CH, CH), 1)).astype(jnp.float32)

    def prefix_excl(oh):
        outs = []
        carry = jnp.zeros((1, N_EXP), jnp.float32)
        for c in range(TOKENS // CH):
            blk = oh[c * CH:(c + 1) * CH]
            outs.append(jnp.dot(tri, blk, preferred_element_type=jnp.float32) + carry)
            carry = carry + jnp.sum(blk, axis=0, keepdims=True)
        return jnp.concatenate(outs, axis=0), carry

    pre0, cnt0 = prefix_excl(oh0)
    pre1, _ = prefix_excl(oh1)
    total = cnt0 + _
    padded = jnp.ceil(total * (1.0 / M)) * M
    tri8x = (lax.broadcasted_iota(jnp.int32, (N_EXP, N_EXP), 0)
             < lax.broadcasted_iota(jnp.int32, (N_EXP, N_EXP), 1)).astype(jnp.float32)
    offs = jnp.dot(padded, tri8x, preferred_element_type=jnp.float32)

    pos0 = jnp.sum(jnp.where(sel1, offs + pre0, 0.0), axis=1, keepdims=True)
    pos1 = jnp.sum(jnp.where(sel2, offs + cnt0 + pre1, 0.0), axis=1, keepdims=True)
    pp_ref[...] = jnp.concatenate([pos0, pos1], axis=1).astype(jnp.int32)

    ntiles = padded * (1.0 / M)
    tri8i = (lax.broadcasted_iota(jnp.int32, (N_EXP, N_EXP), 0)
             <= lax.broadcasted_iota(jnp.int32, (N_EXP, N_EXP), 1)).astype(jnp.float32)
    tend = jnp.dot(ntiles, tri8i, preferred_element_type=jnp.float32)
    iota_t = lax.broadcasted_iota(jnp.int32, (1, TE_W), 1).astype(jnp.float32)
    acc = jnp.zeros((1, TE_W), jnp.float32)
    for e in range(N_EXP):
        acc = acc + (iota_t >= tend[:, e:e + 1]).astype(jnp.float32)
    te = jnp.where(iota_t < tend[:, N_EXP - 1:N_EXP], acc, -1.0)

    present = (total > 0.0).astype(jnp.float32)
    rank = jnp.dot(present, tri8x, preferred_element_type=jnp.float32)
    par_e = rank - 2.0 * jnp.floor(rank * 0.5)
    parity = jnp.zeros((1, TE_W), jnp.float32)
    nxt = jnp.zeros((1, TE_W), jnp.float32)
    for e in range(N_EXP):
        n_e = jnp.float32(-1.0)
        for e2 in range(N_EXP - 1, e, -1):
            n_e = jnp.where(total[0, e2] > 0.0, jnp.float32(e2), n_e)
        sel_e = te == e
        parity = parity + jnp.where(sel_e, par_e[:, e:e + 1], 0.0)
        nxt = nxt + jnp.where(sel_e, n_e, 0.0)
    meta = jnp.concatenate([te, parity, nxt, jnp.zeros((1, TE_W), jnp.float32)], axis=0)
    te_ref[...] = meta.astype(jnp.int32)


def _ffn_body(meta_ref, xs_ref, rw_ref, w1_hbm, b1_ref, w2_hbm, b2_ref, y_ref,
              w1b, w2b, sem1, sem2):
    i = pl.program_id(0)
    te_i = meta_ref[0, i]
    par = meta_ref[1, i]
    nxt = meta_ref[2, i]
    prev = meta_ref[0, jnp.maximum(i - 1, 0)]
    boundary = jnp.logical_or(i == 0, te_i != prev)

    def _start(e, p):
        h1 = D_MODEL // 2
        h2 = D_FF // 2
        pltpu.make_async_copy(w1_hbm.at[e, pl.ds(0, h1)],
                              w1b.at[p, pl.ds(0, h1)], sem1.at[p]).start()
        pltpu.make_async_copy(w1_hbm.at[e, pl.ds(h1, h1)],
                              w1b.at[p, pl.ds(h1, h1)], sem1.at[p]).start()
        pltpu.make_async_copy(w2_hbm.at[e, pl.ds(0, h2)],
                              w2b.at[p, pl.ds(0, h2)], sem2.at[p]).start()
        pltpu.make_async_copy(w2_hbm.at[e, pl.ds(h2, h2)],
                              w2b.at[p, pl.ds(h2, h2)], sem2.at[p]).start()

    @pl.when(jnp.logical_and(te_i >= 0, boundary))
    def _prefetch():
        @pl.when(i == 0)
        def _first():
            _start(te_i, par)

        @pl.when(nxt >= 0)
        def _next():
            _start(nxt, 1 - par)

        pltpu.make_async_copy(w1_hbm.at[0], w1b.at[par], sem1.at[par]).wait()
        pltpu.make_async_copy(w2_hbm.at[0], w2b.at[par], sem2.at[par]).wait()

    @pl.when(te_i >= 0)
    def _compute():
        h = jnp.dot(xs_ref[...], w1b[par], preferred_element_type=jnp.float32) + b1_ref[0]
        h = h * 0.5 * (1.0 + lax.erf(h * 0.7071067811865476))
        o = jnp.dot(h, w2b[par], preferred_element_type=jnp.float32) + b2_ref[0]
        y_ref[...] = o * rw_ref[:, 0:1]


def _dispatch_body(x_hbm, pp_hbm, ww_hbm, xs_hbm, rw_hbm,
                   tok_a, tok_b, ppa, ppb, rwa, rwb, xva, xvb,
                   sga, sgb, ssa, ssb, srw):
    wid = lax.axis_index("s") * 2 + lax.axis_index("c")
    base_tok = wid * TPW
    for j in range(TPW // 16):
        v = lax.iota(jnp.int32, 16) + (16 * j)
        tok_a[pl.ds(16 * j, 16)] = base_tok + (v >> 1)
        tok_b[pl.ds(16 * j, 16)] = base_tok + (TPW // 2) + (v >> 1)
    cp_ga = pltpu.async_copy(x_hbm.at[tok_a], xva, sga)
    cp_gb = pltpu.async_copy(x_hbm.at[tok_b], xvb, sgb)
    pltpu.sync_copy(pp_hbm.at[pl.ds(2 * TPW * wid, TPW)], ppa)
    pltpu.sync_copy(pp_hbm.at[pl.ds(2 * TPW * wid + TPW, TPW)], ppb)
    pltpu.sync_copy(ww_hbm.at[pl.ds(2 * TPW * wid, TPW)], rwa)
    pltpu.sync_copy(ww_hbm.at[pl.ds(2 * TPW * wid + TPW, TPW)], rwb)
    cp_ra = pltpu.async_copy(rwa, rw_hbm.at[ppa], srw)
    cp_rb = pltpu.async_copy(rwb, rw_hbm.at[ppb], srw)
    cp_ga.wait()
    cp_sa = pltpu.async_copy(xva, xs_hbm.at[ppa], ssa)
    cp_gb.wait()
    cp_sb = pltpu.async_copy(xvb, xs_hbm.at[ppb], ssb)
    cp_ra.wait()
    cp_rb.wait()
    cp_sa.wait()
    cp_sb.wait()


_CCH = 4
_CT = TPW // _CCH


def _combine_body(y_hbm, pp_hbm, out_hbm, ppv, g0, g1, ob0, ob1,
                  gs0, gs1, ss0, ss1):
    wid = lax.axis_index("s") * 2 + lax.axis_index("c")
    pltpu.sync_copy(pp_hbm.at[pl.ds(2 * TPW * wid, 2 * TPW)], ppv)
    gbuf = [g0, g1]
    obuf = [ob0, ob1]
    gsem = [gs0, gs1]
    ssem = [ss0, ss1]
    gcp = [None] * _CCH
    scp = [None] * _CCH
    for c in range(2):
        gcp[c] = pltpu.async_copy(
            y_hbm.at[ppv.at[pl.ds(2 * _CT * c, 2 * _CT)]], gbuf[c], gsem[c])
    for c in range(_CCH):
        p = c % 2
        gcp[c].wait()
        if c >= 2:
            scp[c - 2].wait()
        g = gbuf[p]
        ob = obuf[p]

        def body(i, _, g=g, ob=ob):
            for ch in range(D_MODEL // 16):
                a = g[2 * i, pl.ds(16 * ch, 16)]
                b = g[2 * i + 1, pl.ds(16 * ch, 16)]
                ob[i, pl.ds(16 * ch, 16)] = a + b
            return 0

        lax.fori_loop(0, _CT, body, 0)
        scp[c] = pltpu.async_copy(
            ob, out_hbm.at[pl.ds(TPW * wid + _CT * c, _CT)], ssem[p])
        if c + 2 < _CCH:
            gcp[c + 2] = pltpu.async_copy(
                y_hbm.at[ppv.at[pl.ds(2 * _CT * (c + 2), 2 * _CT)]], gbuf[p], gsem[p])
    scp[_CCH - 2].wait()
    scp[_CCH - 1].wait()


@functools.lru_cache(maxsize=1)
def _sc_kernels():
    mesh = plsc.VectorSubcoreMesh(core_axis_name="c", subcore_axis_name="s")
    dispatch = pl.kernel(
        _dispatch_body,
        out_type=(
            jax.ShapeDtypeStruct((P, D_MODEL), jnp.float32),
            jax.ShapeDtypeStruct((P, 128), jnp.float32),
        ),
        mesh=mesh,
        scratch_types=[
            pltpu.VMEM((TPW,), jnp.int32),
            pltpu.VMEM((TPW,), jnp.int32),
            pltpu.VMEM((TPW,), jnp.int32),
            pltpu.VMEM((TPW,), jnp.int32),
            pltpu.VMEM((TPW, 128), jnp.float32),
            pltpu.VMEM((TPW, 128), jnp.float32),
            pltpu.VMEM((TPW, D_MODEL), jnp.float32),
            pltpu.VMEM((TPW, D_MODEL), jnp.float32),
            pltpu.SemaphoreType.DMA,
            pltpu.SemaphoreType.DMA,
            pltpu.SemaphoreType.DMA,
            pltpu.SemaphoreType.DMA,
            pltpu.SemaphoreType.DMA,
        ],
    )
    combine = pl.kernel(
        _combine_body,
        out_type=jax.ShapeDtypeStruct((TOKENS, D_MODEL), jnp.float32),
        mesh=mesh,
        scratch_types=[
            pltpu.VMEM((2 * TPW,), jnp.int32),
            pltpu.VMEM((2 * _CT, D_MODEL), jnp.float32),
            pltpu.VMEM((2 * _CT, D_MODEL), jnp.float32),
            pltpu.VMEM((_CT, D_MODEL), jnp.float32),
            pltpu.VMEM((_CT, D_MODEL), jnp.float32),
            pltpu.SemaphoreType.DMA,
            pltpu.SemaphoreType.DMA,
            pltpu.SemaphoreType.DMA,
            pltpu.SemaphoreType.DMA,
        ],
    )
    return dispatch, combine


@jax.jit
def kernel(x, gate_w, gate_b, w1, b1, w2, b2):
    batch, seq, d_model = x.shape
    x_flat = x.reshape(batch * seq, d_model)

    logits, pp, ww, te = pl.pallas_call(
        _route_body,
        out_shape=[
            jax.ShapeDtypeStruct((TOKENS, N_EXP), jnp.float32),
            jax.ShapeDtypeStruct((TOKENS, 2), jnp.int32),
            jax.ShapeDtypeStruct((2 * TOKENS, 128), jnp.float32),
            jax.ShapeDtypeStruct((4, TE_W), jnp.int32),
        ],
    )(x_flat, gate_w, gate_b.reshape(1, N_EXP))

    dispatch, combine = _sc_kernels()
    pp_flat = pp.reshape(-1)
    xs, rw = dispatch(x_flat, pp_flat, ww)

    grid_spec = pltpu.PrefetchScalarGridSpec(
        num_scalar_prefetch=1,
        grid=(T_MAX,),
        in_specs=[
            pl.BlockSpec((M, D_MODEL), lambda i, m: (jnp.where(m[0, i] >= 0, i, T_MAX), 0)),
            pl.BlockSpec((M, 128), lambda i, m: (jnp.where(m[0, i] >= 0, i, T_MAX), 0)),
            pl.BlockSpec(memory_space=pltpu.MemorySpace.HBM),
            pl.BlockSpec((1, 1, D_FF), lambda i, m: (jnp.maximum(m[0, i], 0), 0, 0)),
            pl.BlockSpec(memory_space=pltpu.MemorySpace.HBM),
            pl.BlockSpec((1, 1, D_MODEL), lambda i, m: (jnp.maximum(m[0, i], 0), 0, 0)),
        ],
        out_specs=pl.BlockSpec((M, D_MODEL), lambda i, m: (jnp.where(m[0, i] >= 0, i, T_MAX), 0)),
        scratch_shapes=[
            pltpu.VMEM((2, D_MODEL, D_FF), jnp.float32),
            pltpu.VMEM((2, D_FF, D_MODEL), jnp.float32),
            pltpu.SemaphoreType.DMA((2,)),
            pltpu.SemaphoreType.DMA((2,)),
        ],
    )
    y = pl.pallas_call(
        _ffn_body,
        grid_spec=grid_spec,
        out_shape=jax.ShapeDtypeStruct((P, D_MODEL), jnp.float32),
        compiler_params=pltpu.CompilerParams(dimension_semantics=("arbitrary",)),
    )(te, xs, rw,
      w1, b1.reshape(N_EXP, 1, D_FF),
      w2, b2.reshape(N_EXP, 1, D_MODEL))

    out = combine(y, pp_flat)
    return out.reshape(batch, seq, d_model), logits
